# baseline (speedup 1.0000x reference)
import functools

import jax
import jax.numpy as jnp
from jax import lax
from jax.experimental import pallas as pl
from jax.experimental.pallas import tpu as pltpu
from jax.experimental.pallas import tpu_sc as plsc

_OFFSET = 0.001

_INFO = plsc.get_sparse_core_info()
_NC, _NS, _L = _INFO.num_cores, _INFO.num_subcores, _INFO.num_lanes
_NW = _NC * _NS

_N = 262144
_D = 256
_ROWS_W = _N // _NW
_CROWS = 128
_NCHUNK = _ROWS_W // _CROWS
_NBUF = 3


def _sc_body(x_hbm, o_hbm, buf, gsem, ssem):
    wid = lax.axis_index("s") * _NC + lax.axis_index("c")
    base = wid * _ROWS_W

    def gather(i, b):
        return pltpu.make_async_copy(
            x_hbm.at[pl.ds(base + i * _CROWS, _CROWS)], buf.at[b], gsem.at[b]
        )

    def scatter(i, b):
        return pltpu.make_async_copy(
            buf.at[b], o_hbm.at[pl.ds(base + i * _CROWS, _CROWS)], ssem.at[b]
        )

    def compute(b):
        @plsc.parallel_loop(0, _CROWS, unroll=2)
        def _(r):
            for c in range(_D // _L):
                x = buf[b, r, pl.ds(c * _L, _L)]
                buf[b, r, pl.ds(c * _L, _L)] = 1.0 / (jnp.abs(x) + _OFFSET)

    for b in range(_NBUF - 1):
        gather(b, b).start()

    _NMAIN = (_NCHUNK // _NBUF) * _NBUF

    def outer(k, carry):
        i0 = k * _NBUF
        for j in range(_NBUF):
            i = i0 + j
            gather(i, j).wait()
            compute(j)
            scatter(i, j).start()
            nb = (j + _NBUF - 1) % _NBUF

            @pl.when(i + _NBUF - 1 < _NCHUNK)
            def _():
                @pl.when(i >= 1)
                def _():
                    scatter(i - 1, nb).wait()

                gather(i + _NBUF - 1, nb).start()

        return carry

    lax.fori_loop(0, _NMAIN // _NBUF, outer, 0)

    lastb = (_NCHUNK - 1) % _NBUF
    gather(_NCHUNK - 1, lastb).wait()
    compute(lastb)
    scatter(_NCHUNK - 1, lastb).start()

    for i in range(_NCHUNK - _NBUF, _NCHUNK):
        scatter(i, i % _NBUF).wait()


_sc_call = functools.partial(
    pl.kernel,
    out_type=jax.ShapeDtypeStruct((_N, _D), jnp.float32),
    mesh=plsc.VectorSubcoreMesh(core_axis_name="c", subcore_axis_name="s"),
    scratch_types=[
        pltpu.VMEM((_NBUF, _CROWS, _D), jnp.float32),
        pltpu.SemaphoreType.DMA((_NBUF,)),
        pltpu.SemaphoreType.DMA((_NBUF,)),
    ],
    compiler_params=pltpu.CompilerParams(use_tc_tiling_on_sc=True),
)(_sc_body)


def kernel(xyz):
    return _sc_call(xyz)


# speedup vs baseline: 1.0030x; 1.0030x over previous
import functools

import jax
import jax.numpy as jnp
from jax import lax
from jax.experimental import pallas as pl
from jax.experimental.pallas import tpu as pltpu
from jax.experimental.pallas import tpu_sc as plsc

_OFFSET = 0.001

_INFO = plsc.get_sparse_core_info()
_NC, _NS, _L = _INFO.num_cores, _INFO.num_subcores, _INFO.num_lanes
_NW = _NC * _NS

_N = 262144
_D = 256
_ROWS_W = _N // _NW
_CROWS = 64
_NCHUNK = _ROWS_W // _CROWS
_NBUF = 4


def _sc_body(x_hbm, o_hbm, buf, gsem, ssem):
    wid = lax.axis_index("s") * _NC + lax.axis_index("c")
    base = wid * _ROWS_W

    def gather(i, b):
        return pltpu.make_async_copy(
            x_hbm.at[pl.ds(base + i * _CROWS, _CROWS)], buf.at[b], gsem.at[b]
        )

    def scatter(i, b):
        return pltpu.make_async_copy(
            buf.at[b], o_hbm.at[pl.ds(base + i * _CROWS, _CROWS)], ssem.at[b]
        )

    for b in range(_NBUF - 1):
        gather(b, b).start()

    def outer(k, carry):
        i0 = k * _NBUF
        for j in range(_NBUF):
            i = i0 + j
            gather(i, j).wait()

            @plsc.parallel_loop(0, _CROWS, unroll=2)
            def _(r):
                for c in range(_D // _L):
                    x = buf[j, r, pl.ds(c * _L, _L)]
                    buf[j, r, pl.ds(c * _L, _L)] = 1.0 / (jnp.abs(x) + _OFFSET)

            scatter(i, j).start()
            nb = (j + _NBUF - 1) % _NBUF

            @pl.when(i + _NBUF - 1 < _NCHUNK)
            def _():
                @pl.when(i >= 1)
                def _():
                    scatter(i - 1, nb).wait()

                gather(i + _NBUF - 1, nb).start()

        return carry

    lax.fori_loop(0, _NCHUNK // _NBUF, outer, 0)

    for j in range(_NBUF):
        scatter(_NCHUNK - _NBUF + j, j).wait()


_sc_call = functools.partial(
    pl.kernel,
    out_type=jax.ShapeDtypeStruct((_N, _D), jnp.float32),
    mesh=plsc.VectorSubcoreMesh(core_axis_name="c", subcore_axis_name="s"),
    scratch_types=[
        pltpu.VMEM((_NBUF, _CROWS, _D), jnp.float32),
        pltpu.SemaphoreType.DMA((_NBUF,)),
        pltpu.SemaphoreType.DMA((_NBUF,)),
    ],
    compiler_params=pltpu.CompilerParams(use_tc_tiling_on_sc=True),
)(_sc_body)


def kernel(xyz):
    return _sc_call(xyz)


# speedup vs baseline: 1.0092x; 1.0061x over previous
import functools

import jax
import jax.numpy as jnp
from jax import lax
from jax.experimental import pallas as pl
from jax.experimental.pallas import tpu as pltpu
from jax.experimental.pallas import tpu_sc as plsc

_OFFSET = 0.001

_INFO = plsc.get_sparse_core_info()
_NC, _NS, _L = _INFO.num_cores, _INFO.num_subcores, _INFO.num_lanes
_NW = _NC * _NS

_N = 262144
_D = 256
_ROWS_W = _N // _NW
_CROWS = 64
_NCHUNK = _ROWS_W // _CROWS
_NBUF = 4


def _sc_body(x_hbm, o_hbm, buf, gsem, ssem):
    wid = lax.axis_index("c") * _NS + lax.axis_index("s")
    base = wid * _ROWS_W

    def gather(i, b):
        return pltpu.make_async_copy(
            x_hbm.at[pl.ds(base + i * _CROWS, _CROWS)], buf.at[b], gsem.at[b]
        )

    def scatter(i, b):
        return pltpu.make_async_copy(
            buf.at[b], o_hbm.at[pl.ds(base + i * _CROWS, _CROWS)], ssem.at[b]
        )

    for b in range(_NBUF - 1):
        gather(b, b).start()

    def outer(k, carry):
        i0 = k * _NBUF
        for j in range(_NBUF):
            i = i0 + j
            gather(i, j).wait()

            @plsc.parallel_loop(0, _CROWS, unroll=2)
            def _(r):
                for c in range(_D // _L):
                    x = buf[j, r, pl.ds(c * _L, _L)]
                    buf[j, r, pl.ds(c * _L, _L)] = 1.0 / (jnp.abs(x) + _OFFSET)

            scatter(i, j).start()
            nb = (j + _NBUF - 1) % _NBUF

            @pl.when(i + _NBUF - 1 < _NCHUNK)
            def _():
                @pl.when(i >= 1)
                def _():
                    scatter(i - 1, nb).wait()

                gather(i + _NBUF - 1, nb).start()

        return carry

    lax.fori_loop(0, _NCHUNK // _NBUF, outer, 0)

    for j in range(_NBUF):
        scatter(_NCHUNK - _NBUF + j, j).wait()


_sc_call = functools.partial(
    pl.kernel,
    out_type=jax.ShapeDtypeStruct((_N, _D), jnp.float32),
    mesh=plsc.VectorSubcoreMesh(core_axis_name="c", subcore_axis_name="s"),
    scratch_types=[
        pltpu.VMEM((_NBUF, _CROWS, _D), jnp.float32),
        pltpu.SemaphoreType.DMA((_NBUF,)),
        pltpu.SemaphoreType.DMA((_NBUF,)),
    ],
    compiler_params=pltpu.CompilerParams(use_tc_tiling_on_sc=True),
)(_sc_body)


def kernel(xyz):
    return _sc_call(xyz)


# speedup vs baseline: 1.0105x; 1.0013x over previous
import functools

import jax
import jax.numpy as jnp
from jax import lax
from jax.experimental import pallas as pl
from jax.experimental.pallas import tpu as pltpu
from jax.experimental.pallas import tpu_sc as plsc

_OFFSET = 0.001

_INFO = plsc.get_sparse_core_info()
_NC, _NS, _L = _INFO.num_cores, _INFO.num_subcores, _INFO.num_lanes
_NW = _NC * _NS

_N = 262144
_D = 256
_ROWS_W = _N // _NW
_CROWS = 32
_NCHUNK = _ROWS_W // _CROWS
_NBUF = 8


def _sc_body(x_hbm, o_hbm, buf, gsem, ssem):
    wid = lax.axis_index("c") * _NS + lax.axis_index("s")
    base = wid * _ROWS_W

    def gather(i, b):
        return pltpu.make_async_copy(
            x_hbm.at[pl.ds(base + i * _CROWS, _CROWS)], buf.at[b], gsem.at[b]
        )

    def scatter(i, b):
        return pltpu.make_async_copy(
            buf.at[b], o_hbm.at[pl.ds(base + i * _CROWS, _CROWS)], ssem.at[b]
        )

    for b in range(_NBUF - 1):
        gather(b, b).start()

    def outer(k, carry):
        i0 = k * _NBUF
        for j in range(_NBUF):
            i = i0 + j
            gather(i, j).wait()

            @plsc.parallel_loop(0, _CROWS, unroll=2)
            def _(r):
                for c in range(_D // _L):
                    x = buf[j, r, pl.ds(c * _L, _L)]
                    buf[j, r, pl.ds(c * _L, _L)] = 1.0 / (jnp.abs(x) + _OFFSET)

            scatter(i, j).start()
            nb = (j + _NBUF - 1) % _NBUF

            @pl.when(i + _NBUF - 1 < _NCHUNK)
            def _():
                @pl.when(i >= 1)
                def _():
                    scatter(i - 1, nb).wait()

                gather(i + _NBUF - 1, nb).start()

        return carry

    lax.fori_loop(0, _NCHUNK // _NBUF, outer, 0)

    for j in range(_NBUF):
        scatter(_NCHUNK - _NBUF + j, j).wait()


_sc_call = functools.partial(
    pl.kernel,
    out_type=jax.ShapeDtypeStruct((_N, _D), jnp.float32),
    mesh=plsc.VectorSubcoreMesh(core_axis_name="c", subcore_axis_name="s"),
    scratch_types=[
        pltpu.VMEM((_NBUF, _CROWS, _D), jnp.float32),
        pltpu.SemaphoreType.DMA((_NBUF,)),
        pltpu.SemaphoreType.DMA((_NBUF,)),
    ],
    compiler_params=pltpu.CompilerParams(use_tc_tiling_on_sc=True),
)(_sc_body)


def kernel(xyz):
    return _sc_call(xyz)
